# Initial kernel scaffold; baseline (speedup 1.0000x reference)
import jax
import jax.numpy as jnp
from jax.experimental import pallas as pl


def kernel(input_ids, fix_vec, fix_offsets, table):
    raise NotImplementedError("write your pallas kernel here")


# speedup vs baseline: 5.3937x; 5.3937x over previous
import functools

import jax
import jax.numpy as jnp
from jax import lax
from jax.experimental import pallas as pl
from jax.experimental.pallas import tpu as pltpu
from jax.experimental.pallas import tpu_sc as plsc

B, L, D, E = 4096, 77, 64, 8
NW = 32
IDS_PER_W = (B * L) // NW
CHUNK = 112
NCHUNK = IDS_PER_W // CHUNK
SCAT = 128
NSCAT = (B * E) // (NW * SCAT)


def kernel(input_ids, fix_vec, fix_offsets, table):
    ids_r = input_ids.reshape(NW, NCHUNK, CHUNK)
    pos = (jnp.arange(B, dtype=jnp.int32) * L + fix_offsets + 1)[:, None] \
        + jnp.arange(E, dtype=jnp.int32)[None, :]
    pos_r = pos.reshape(NW, NSCAT, SCAT)
    fix_tiled = jnp.tile(fix_vec, (SCAT // E, 1))

    mesh = plsc.VectorSubcoreMesh(core_axis_name="c", subcore_axis_name="s")

    @functools.partial(
        pl.kernel, mesh=mesh,
        compiler_params=pltpu.CompilerParams(use_tc_tiling_on_sc=False),
        out_type=jax.ShapeDtypeStruct((B * L, D), jnp.float32),
        scratch_types=[
            pltpu.VMEM((NCHUNK, CHUNK), jnp.int32),
            pltpu.VMEM((CHUNK, D), jnp.float32),
            pltpu.VMEM((NSCAT, SCAT), jnp.int32),
            pltpu.VMEM((SCAT, D), jnp.float32),
            pltpu.SemaphoreType.DMA,
        ],
    )
    def emb_fix_kernel(ids_hbm, pos_hbm, fixt_hbm, table_hbm, out_hbm,
                       idx_v, rows_v, pos_v, fixt_v, sem):
        wid = lax.axis_index("s") * 2 + lax.axis_index("c")
        pltpu.sync_copy(ids_hbm.at[wid], idx_v)
        base = wid * IDS_PER_W

        @pl.loop(0, NCHUNK)
        def _(j):
            pltpu.async_copy(table_hbm.at[idx_v.at[j]], rows_v, sem).wait()
            pltpu.sync_copy(rows_v, out_hbm.at[pl.ds(base + j * CHUNK, CHUNK)])

        pltpu.sync_copy(pos_hbm.at[wid], pos_v)
        pltpu.sync_copy(fixt_hbm, fixt_v)

        @pl.loop(0, NSCAT)
        def _(j):
            pltpu.sync_copy(fixt_v, out_hbm.at[pos_v.at[j]])

    out = emb_fix_kernel(ids_r, pos_r, fix_tiled, table)
    return out.reshape(B, L, D)
